# baseline (speedup 1.0000x reference)
import jax
import jax.numpy as jnp
from jax.experimental import pallas as pl
from jax.experimental.pallas import tpu as pltpu

_ALPHA = 0.3
_LANE = 128


def _gat_kernel(x_ref, w_ref, b_ref, alr_ref, out_ref, g_ref, s_ref,
                *, num_heads, c_head):
    x = x_ref[0]
    N = x.shape[0]
    nrm = jnp.sqrt(jnp.sum(x * x, axis=1, keepdims=True))
    n = x / jnp.maximum(nrm, 1e-12)
    xx = jax.lax.dot_general(n, n, (((1,), (1,)), ((), ())),
                             preferred_element_type=jnp.float32)
    maskb = xx.astype(jnp.bfloat16) > jnp.bfloat16(0.0)
    feats = jax.lax.dot_general(x, w_ref[...], (((1,), (1,)), ((), ())),
                                preferred_element_type=jnp.float32)
    feats = feats + b_ref[...][None, :]
    ones_blk = jnp.ones((N, _LANE - c_head), jnp.bfloat16)
    for h in range(num_heads):
        g_ref[:, h * _LANE:h * _LANE + c_head] = (
            feats[:, h * c_head:(h + 1) * c_head].astype(jnp.bfloat16))
        g_ref[:, h * _LANE + c_head:(h + 1) * _LANE] = ones_blk
    H = num_heads
    alr = jnp.dot(feats, alr_ref[...],
                  preferred_element_type=jnp.float32)
    lcol = alr[:, :H]
    rcol = alr[:, H:]
    rrow = rcol.T
    zero = jnp.bfloat16(0.0)
    mrv = jnp.max(rcol, axis=0, keepdims=True)
    t1 = lcol + mrv
    qe = jnp.exp(jnp.minimum((_ALPHA - 1.0) * t1, 60.0)
                 ).astype(jnp.bfloat16)
    rsh = rrow - mrv.T
    era = jnp.exp(rsh).astype(jnp.bfloat16)
    eara = jnp.exp(_ALPHA * rsh).astype(jnp.bfloat16)
    for h in range(num_heads):
        P = jnp.where(maskb,
                      jnp.maximum(era[h:h + 1, :],
                                  qe[:, h:h + 1] * eara[h:h + 1, :]),
                      zero)
        s_ref[:, h * _LANE:(h + 1) * _LANE] = jnp.dot(
            P, g_ref[:, h * _LANE:(h + 1) * _LANE],
            preferred_element_type=jnp.float32)
    dcols = jnp.concatenate(
        [s_ref[:, h * _LANE + c_head:h * _LANE + c_head + 1]
         for h in range(num_heads)], axis=1)
    inv = 1.0 / dcols
    for h in range(num_heads):
        out_ref[0, :, h * c_head:(h + 1) * c_head] = (
            s_ref[:, h * _LANE:h * _LANE + c_head] * inv[:, h:h + 1])


def kernel(node_feats, W, b, a):
    B, N, C = node_feats.shape
    H = a.shape[0]
    c_head = a.shape[1] // 2
    O = H * c_head
    eye = jnp.eye(H, dtype=a.dtype)
    Al = (a[:, :c_head, None] * eye[:, None, :]).reshape(O, H)
    Ar = (a[:, c_head:, None] * eye[:, None, :]).reshape(O, H)
    Alr = jnp.concatenate([Al, Ar], axis=1)

    grid = (B,)
    out = pl.pallas_call(
        lambda *refs: _gat_kernel(*refs, num_heads=H, c_head=c_head),
        grid=grid,
        in_specs=[
            pl.BlockSpec((1, N, C), lambda i: (i, 0, 0)),
            pl.BlockSpec((O, C), lambda i: (0, 0)),
            pl.BlockSpec((O,), lambda i: (0,)),
            pl.BlockSpec((O, 2 * H), lambda i: (0, 0)),
        ],
        out_specs=pl.BlockSpec((1, N, O), lambda i: (i, 0, 0)),
        out_shape=jax.ShapeDtypeStruct((B, N, O), jnp.float32),
        scratch_shapes=[
            pltpu.VMEM((N, H * _LANE), jnp.bfloat16),
            pltpu.VMEM((N, H * _LANE), jnp.float32),
        ],
        compiler_params=pltpu.CompilerParams(
            dimension_semantics=("parallel",)),
    )(node_feats, W, b, Alr)
    return out


# speedup vs baseline: 1.2267x; 1.2267x over previous
import jax
import jax.numpy as jnp
from jax.experimental import pallas as pl
from jax.experimental.pallas import tpu as pltpu

_ALPHA = 0.3
_LANE = 128


def _gat_kernel(x_ref, w_ref, b_ref, alr_ref, out_ref, g_ref,
                *, num_heads, c_head):
    x = x_ref[0]
    N = x.shape[0]
    nrm = jnp.sqrt(jnp.sum(x * x, axis=1, keepdims=True))
    n = x / jnp.maximum(nrm, 1e-12)
    xx = jax.lax.dot_general(n, n, (((1,), (1,)), ((), ())),
                             preferred_element_type=jnp.float32)
    maskb = xx.astype(jnp.bfloat16) > jnp.bfloat16(0.0)
    feats = jax.lax.dot_general(x, w_ref[...], (((1,), (1,)), ((), ())),
                                preferred_element_type=jnp.float32)
    feats = feats + b_ref[...][None, :]
    ones_blk = jnp.ones((N, _LANE - c_head), jnp.bfloat16)
    for h in range(num_heads):
        g_ref[:, h * _LANE:h * _LANE + c_head] = (
            feats[:, h * c_head:(h + 1) * c_head].astype(jnp.bfloat16))
        g_ref[:, h * _LANE + c_head:(h + 1) * _LANE] = ones_blk
    H = num_heads
    alr = jnp.dot(feats, alr_ref[...],
                  preferred_element_type=jnp.float32)
    lcol = alr[:, :H]
    rcol = alr[:, H:]
    rrow = rcol.T
    zero = jnp.bfloat16(0.0)
    mrv = jnp.max(rcol, axis=0, keepdims=True)
    t1 = lcol + mrv
    qe = jnp.exp(jnp.minimum((_ALPHA - 1.0) * t1, 60.0)
                 ).astype(jnp.bfloat16)
    rsh = rrow - mrv.T
    era = jnp.exp(rsh).astype(jnp.bfloat16)
    eara = jnp.exp(_ALPHA * rsh).astype(jnp.bfloat16)
    for h in range(num_heads):
        P = jnp.where(maskb,
                      jnp.maximum(era[h:h + 1, :],
                                  qe[:, h:h + 1] * eara[h:h + 1, :]),
                      zero)
        AG = jnp.dot(P, g_ref[:, h * _LANE:(h + 1) * _LANE],
                     preferred_element_type=jnp.float32)
        out_ref[0, :, h * c_head:(h + 1) * c_head] = (
            AG[:, :c_head] / AG[:, c_head:c_head + 1])


def kernel(node_feats, W, b, a):
    B, N, C = node_feats.shape
    H = a.shape[0]
    c_head = a.shape[1] // 2
    O = H * c_head
    eye = jnp.eye(H, dtype=a.dtype)
    Al = (a[:, :c_head, None] * eye[:, None, :]).reshape(O, H)
    Ar = (a[:, c_head:, None] * eye[:, None, :]).reshape(O, H)
    Alr = jnp.concatenate([Al, Ar], axis=1)

    grid = (B,)
    out = pl.pallas_call(
        lambda *refs: _gat_kernel(*refs, num_heads=H, c_head=c_head),
        grid=grid,
        in_specs=[
            pl.BlockSpec((1, N, C), lambda i: (i, 0, 0)),
            pl.BlockSpec((O, C), lambda i: (0, 0)),
            pl.BlockSpec((O,), lambda i: (0,)),
            pl.BlockSpec((O, 2 * H), lambda i: (0, 0)),
        ],
        out_specs=pl.BlockSpec((1, N, O), lambda i: (i, 0, 0)),
        out_shape=jax.ShapeDtypeStruct((B, N, O), jnp.float32),
        scratch_shapes=[
            pltpu.VMEM((N, H * _LANE), jnp.bfloat16),
        ],
        compiler_params=pltpu.CompilerParams(
            dimension_semantics=("parallel",)),
    )(node_feats, W, b, Alr)
    return out


# speedup vs baseline: 1.2638x; 1.0302x over previous
import jax
import jax.numpy as jnp
from jax.experimental import pallas as pl
from jax.experimental.pallas import tpu as pltpu

_ALPHA = 0.3
_LANE = 128


def _gat_kernel(x_ref, w_ref, b_ref, al_ref, ar_ref, out_ref, g_ref,
                *, num_heads, c_head):
    x = x_ref[0]
    N = x.shape[0]
    nrm = jnp.sqrt(jnp.sum(x * x, axis=1, keepdims=True))
    n = x / jnp.maximum(nrm, 1e-12)
    xx = jax.lax.dot_general(n, n, (((1,), (1,)), ((), ())),
                             preferred_element_type=jnp.float32)
    maskb = xx.astype(jnp.bfloat16) > jnp.bfloat16(0.0)
    feats = jax.lax.dot_general(x, w_ref[...], (((1,), (1,)), ((), ())),
                                preferred_element_type=jnp.float32)
    feats = feats + b_ref[...][None, :]
    ones_blk = jnp.ones((N, _LANE - c_head), jnp.bfloat16)
    for h in range(num_heads):
        g_ref[:, h * _LANE:h * _LANE + c_head] = (
            feats[:, h * c_head:(h + 1) * c_head].astype(jnp.bfloat16))
        g_ref[:, h * _LANE + c_head:(h + 1) * _LANE] = ones_blk
    lcol = jnp.dot(feats, al_ref[...],
                   preferred_element_type=jnp.float32)
    rcol = jnp.dot(feats, ar_ref[...],
                   preferred_element_type=jnp.float32)
    rrow = rcol.T
    zero = jnp.bfloat16(0.0)
    mrv = jnp.max(rcol, axis=0, keepdims=True)
    t1 = lcol + mrv
    qe = jnp.exp(jnp.minimum((_ALPHA - 1.0) * t1, 60.0)
                 ).astype(jnp.bfloat16)
    rsh = rrow - mrv.T
    era = jnp.exp(rsh).astype(jnp.bfloat16)
    eara = jnp.exp(_ALPHA * rsh).astype(jnp.bfloat16)
    for h in range(num_heads):
        P = jnp.where(maskb,
                      jnp.maximum(era[h:h + 1, :],
                                  qe[:, h:h + 1] * eara[h:h + 1, :]),
                      zero)
        AG = jnp.dot(P, g_ref[:, h * _LANE:(h + 1) * _LANE],
                     preferred_element_type=jnp.float32)
        out_ref[0, :, h * c_head:(h + 1) * c_head] = (
            AG[:, :c_head] / AG[:, c_head:c_head + 1])


def kernel(node_feats, W, b, a):
    B, N, C = node_feats.shape
    H = a.shape[0]
    c_head = a.shape[1] // 2
    O = H * c_head
    eye = jnp.eye(H, dtype=a.dtype)
    Al = (a[:, :c_head, None] * eye[:, None, :]).reshape(O, H)
    Ar = (a[:, c_head:, None] * eye[:, None, :]).reshape(O, H)

    grid = (B,)
    out = pl.pallas_call(
        lambda *refs: _gat_kernel(*refs, num_heads=H, c_head=c_head),
        grid=grid,
        in_specs=[
            pl.BlockSpec((1, N, C), lambda i: (i, 0, 0)),
            pl.BlockSpec((O, C), lambda i: (0, 0)),
            pl.BlockSpec((O,), lambda i: (0,)),
            pl.BlockSpec((O, H), lambda i: (0, 0)),
            pl.BlockSpec((O, H), lambda i: (0, 0)),
        ],
        out_specs=pl.BlockSpec((1, N, O), lambda i: (i, 0, 0)),
        out_shape=jax.ShapeDtypeStruct((B, N, O), jnp.float32),
        scratch_shapes=[
            pltpu.VMEM((N, H * _LANE), jnp.bfloat16),
        ],
        compiler_params=pltpu.CompilerParams(
            dimension_semantics=("parallel",)),
    )(node_feats, W, b, Al, Ar)
    return out
